# initial kernel scaffold (unmeasured)
import jax
import jax.numpy as jnp
from jax import lax
from jax.experimental import pallas as pl
from jax.experimental.pallas import tpu as pltpu

N_DEV = 8
B, S, C = 4, 2048, 1024
TAPS = 4
CHUNK = S // N_DEV


def kernel(x, k, Wp):
    def body(x_ref, k_ref, w_ref, out_ref, recv_buf, send_sem, recv_sem, credit_sem):
        my = lax.axis_index("i")
        left = lax.rem(my + N_DEV - 1, N_DEV)
        right = lax.rem(my + 1, N_DEV)

        barrier = pltpu.get_barrier_semaphore()
        for nbr in (left, right):
            pl.semaphore_signal(
                barrier, inc=1, device_id=(nbr,),
                device_id_type=pl.DeviceIdType.MESH,
            )
        pl.semaphore_wait(barrier, 2)

        pl.semaphore_signal(
            credit_sem, inc=1, device_id=(left,),
            device_id_type=pl.DeviceIdType.MESH,
        )

        for c in range(N_DEV):
            base = c * CHUNK
            acc = x_ref[:, pl.ds(base, CHUNK), :] * k_ref[TAPS - 1, :][None, None, :]
            for d in range(1, TAPS):
                tap = k_ref[TAPS - 1 - d, :][None, None, :]
                if base - d >= 0:
                    xs = x_ref[:, pl.ds(base - d, CHUNK), :]
                else:
                    xs = jnp.concatenate(
                        [
                            jnp.zeros((B, d, C), jnp.float32),
                            x_ref[:, : CHUNK - d, :],
                        ],
                        axis=1,
                    )
                acc = acc + xs * tap
            a = acc * jax.nn.sigmoid(acc)
            for b in range(B):
                out_ref[b, pl.ds(base, CHUNK), :] = jnp.dot(
                    a[b], w_ref[:, :], preferred_element_type=jnp.float32
                )

        total_steps = 2 * (N_DEV - 1)
        step = 0
        for phase in range(2):
            for s in range(N_DEV - 1):
                if phase == 0:
                    send_c = lax.rem(my - s + 2 * N_DEV, N_DEV)
                    recv_c = lax.rem(my - s - 1 + 2 * N_DEV, N_DEV)
                else:
                    send_c = lax.rem(my + 1 - s + 2 * N_DEV, N_DEV)
                    recv_c = lax.rem(my - s + 2 * N_DEV, N_DEV)
                send_base = send_c * CHUNK
                recv_base = recv_c * CHUNK

                pl.semaphore_wait(credit_sem, 1)
                rdma = pltpu.make_async_remote_copy(
                    src_ref=out_ref.at[:, pl.ds(send_base, CHUNK), :],
                    dst_ref=recv_buf,
                    send_sem=send_sem,
                    recv_sem=recv_sem,
                    device_id=(right,),
                    device_id_type=pl.DeviceIdType.MESH,
                )
                rdma.start()
                rdma.wait()

                cur = recv_buf[:, :, :]
                if phase == 0:
                    out_ref[:, pl.ds(recv_base, CHUNK), :] = (
                        out_ref[:, pl.ds(recv_base, CHUNK), :] + cur
                    )
                else:
                    out_ref[:, pl.ds(recv_base, CHUNK), :] = cur

                step += 1
                if step < total_steps:
                    pl.semaphore_signal(
                        credit_sem, inc=1, device_id=(left,),
                        device_id_type=pl.DeviceIdType.MESH,
                    )

    return pl.pallas_call(
        body,
        out_shape=jax.ShapeDtypeStruct((B, S, C), jnp.float32),
        in_specs=[
            pl.BlockSpec(memory_space=pltpu.VMEM),
            pl.BlockSpec(memory_space=pltpu.VMEM),
            pl.BlockSpec(memory_space=pltpu.VMEM),
        ],
        out_specs=pl.BlockSpec(memory_space=pltpu.VMEM),
        scratch_shapes=[
            pltpu.VMEM((B, CHUNK, C), jnp.float32),
            pltpu.SemaphoreType.DMA,
            pltpu.SemaphoreType.DMA,
            pltpu.SemaphoreType.REGULAR,
        ],
        compiler_params=pltpu.CompilerParams(collective_id=0),
    )(x, k, Wp)


# baseline (device time: 845222 ns/iter reference)
import jax
import jax.numpy as jnp
from jax import lax
from jax.experimental import pallas as pl
from jax.experimental.pallas import tpu as pltpu

N_DEV = 8
B, S, C = 4, 2048, 1024
TAPS = 4
CHUNK = S // N_DEV
CC = 128
HALO = 8


def kernel(x, k, Wp):
    def body(
        x_ref, k_ref, w_ref, out_ref,
        xbuf, obuf, acc_buf, recv_buf,
        copy_sem, store_sem, send_sem, recv_sem, credit_sem,
    ):
        my = lax.axis_index("i")
        left = lax.rem(my + N_DEV - 1, N_DEV)
        right = lax.rem(my + 1, N_DEV)

        barrier = pltpu.get_barrier_semaphore()
        for nbr in (left, right):
            pl.semaphore_signal(
                barrier, inc=1, device_id=(nbr,),
                device_id_type=pl.DeviceIdType.MESH,
            )
        pl.semaphore_wait(barrier, 2)

        pl.semaphore_signal(
            credit_sem, inc=1, device_id=(left,),
            device_id_type=pl.DeviceIdType.MESH,
        )

        for c in range(S // CC):
            base = c * CC
            if c == 0:
                xbuf[:, :HALO, :] = jnp.zeros((B, HALO, C), jnp.float32)
                cp = pltpu.make_async_copy(
                    x_ref.at[:, pl.ds(0, CC), :],
                    xbuf.at[:, pl.ds(HALO, CC), :],
                    copy_sem,
                )
            else:
                cp = pltpu.make_async_copy(
                    x_ref.at[:, pl.ds(base - HALO, CC + HALO), :],
                    xbuf,
                    copy_sem,
                )
            cp.start()
            cp.wait()
            acc = xbuf[:, pl.ds(HALO, CC), :] * k_ref[TAPS - 1, :][None, None, :]
            for d in range(1, TAPS):
                tap = k_ref[TAPS - 1 - d, :][None, None, :]
                acc = acc + xbuf[:, pl.ds(HALO - d, CC), :] * tap
            a = acc * jax.nn.sigmoid(acc)
            for b in range(B):
                obuf[b, :, :] = jnp.dot(
                    a[b], w_ref[:, :], preferred_element_type=jnp.float32
                )
            st = pltpu.make_async_copy(
                obuf, out_ref.at[:, pl.ds(base, CC), :], store_sem
            )
            st.start()
            st.wait()

        total_steps = 2 * (N_DEV - 1)
        step = 0
        for phase in range(2):
            for s in range(N_DEV - 1):
                if phase == 0:
                    send_c = lax.rem(my - s + 2 * N_DEV, N_DEV)
                    recv_c = lax.rem(my - s - 1 + 2 * N_DEV, N_DEV)
                else:
                    send_c = lax.rem(my + 1 - s + 2 * N_DEV, N_DEV)
                    recv_c = lax.rem(my - s + 2 * N_DEV, N_DEV)
                send_base = pl.multiple_of(send_c * CHUNK, CHUNK)
                recv_base = pl.multiple_of(recv_c * CHUNK, CHUNK)

                pl.semaphore_wait(credit_sem, 1)
                rdma = pltpu.make_async_remote_copy(
                    src_ref=out_ref.at[:, pl.ds(send_base, CHUNK), :],
                    dst_ref=recv_buf,
                    send_sem=send_sem,
                    recv_sem=recv_sem,
                    device_id=(right,),
                    device_id_type=pl.DeviceIdType.MESH,
                )
                rdma.start()
                rdma.wait()

                if phase == 0:
                    ld = pltpu.make_async_copy(
                        out_ref.at[:, pl.ds(recv_base, CHUNK), :],
                        acc_buf, copy_sem,
                    )
                    ld.start()
                    ld.wait()
                    acc_buf[:, :, :] = acc_buf[:, :, :] + recv_buf[:, :, :]
                    st = pltpu.make_async_copy(
                        acc_buf, out_ref.at[:, pl.ds(recv_base, CHUNK), :],
                        store_sem,
                    )
                else:
                    st = pltpu.make_async_copy(
                        recv_buf, out_ref.at[:, pl.ds(recv_base, CHUNK), :],
                        store_sem,
                    )
                st.start()
                st.wait()

                step += 1
                if step < total_steps:
                    pl.semaphore_signal(
                        credit_sem, inc=1, device_id=(left,),
                        device_id_type=pl.DeviceIdType.MESH,
                    )

    return pl.pallas_call(
        body,
        out_shape=jax.ShapeDtypeStruct((B, S, C), jnp.float32),
        in_specs=[
            pl.BlockSpec(memory_space=pl.ANY),
            pl.BlockSpec(memory_space=pltpu.VMEM),
            pl.BlockSpec(memory_space=pltpu.VMEM),
        ],
        out_specs=pl.BlockSpec(memory_space=pl.ANY),
        scratch_shapes=[
            pltpu.VMEM((B, CC + HALO, C), jnp.float32),
            pltpu.VMEM((B, CC, C), jnp.float32),
            pltpu.VMEM((B, CHUNK, C), jnp.float32),
            pltpu.VMEM((B, CHUNK, C), jnp.float32),
            pltpu.SemaphoreType.DMA,
            pltpu.SemaphoreType.DMA,
            pltpu.SemaphoreType.DMA,
            pltpu.SemaphoreType.DMA,
            pltpu.SemaphoreType.REGULAR,
        ],
        compiler_params=pltpu.CompilerParams(collective_id=0),
    )(x, k, Wp)


# device time: 525137 ns/iter; 1.6095x vs baseline; 1.6095x over previous
import jax
import jax.numpy as jnp
from jax import lax
from jax.experimental import pallas as pl
from jax.experimental.pallas import tpu as pltpu

N_DEV = 8
B, S, C = 4, 2048, 1024
TAPS = 4
CHUNK = S // N_DEV
HB = B // 2
CC = 128
HALO = 8
RING_B = True


def kernel(x, k, Wp):
    def body(
        x_ref, k_ref, w_ref, out_ref,
        xbuf, obuf, acc_a, acc_b, recv_a, recv_b,
        copy_sem, store_sem,
        send_sem_a, recv_sem_a, send_sem_b, recv_sem_b,
        credit_a, credit_b,
    ):
        my = lax.axis_index("i")
        left = lax.rem(my + N_DEV - 1, N_DEV)
        right = lax.rem(my + 1, N_DEV)

        barrier = pltpu.get_barrier_semaphore()
        for nbr in (left, right):
            pl.semaphore_signal(
                barrier, inc=1, device_id=(nbr,),
                device_id_type=pl.DeviceIdType.MESH,
            )
        pl.semaphore_wait(barrier, 2)

        pl.semaphore_signal(
            credit_a, inc=1, device_id=(left,),
            device_id_type=pl.DeviceIdType.MESH,
        )
        if RING_B:
            pl.semaphore_signal(
                credit_b, inc=1, device_id=(right,),
                device_id_type=pl.DeviceIdType.MESH,
            )

        for c in range(S // CC):
            base = c * CC
            if c == 0:
                xbuf[:, :HALO, :] = jnp.zeros((B, HALO, C), jnp.float32)
                cp = pltpu.make_async_copy(
                    x_ref.at[:, pl.ds(0, CC), :],
                    xbuf.at[:, pl.ds(HALO, CC), :],
                    copy_sem,
                )
            else:
                cp = pltpu.make_async_copy(
                    x_ref.at[:, pl.ds(base - HALO, CC + HALO), :],
                    xbuf,
                    copy_sem,
                )
            cp.start()
            cp.wait()
            acc = xbuf[:, pl.ds(HALO, CC), :] * k_ref[TAPS - 1, :][None, None, :]
            for d in range(1, TAPS):
                tap = k_ref[TAPS - 1 - d, :][None, None, :]
                acc = acc + xbuf[:, pl.ds(HALO - d, CC), :] * tap
            a = acc * jax.nn.sigmoid(acc)
            for b in range(B):
                obuf[b, :, :] = jnp.dot(
                    a[b], w_ref[:, :], preferred_element_type=jnp.float32
                )
            st = pltpu.make_async_copy(
                obuf, out_ref.at[:, pl.ds(base, CC), :], store_sem
            )
            st.start()
            st.wait()

        total_steps = 2 * (N_DEV - 1)
        step = 0
        for phase in range(2):
            for s in range(N_DEV - 1):
                if phase == 0:
                    send_ca = lax.rem(my - s + 2 * N_DEV, N_DEV)
                    recv_ca = lax.rem(my - s - 1 + 2 * N_DEV, N_DEV)
                    send_cb = lax.rem(my + s + 2 * N_DEV, N_DEV)
                    recv_cb = lax.rem(my + s + 1 + 2 * N_DEV, N_DEV)
                else:
                    send_ca = lax.rem(my + 1 - s + 2 * N_DEV, N_DEV)
                    recv_ca = lax.rem(my - s + 2 * N_DEV, N_DEV)
                    send_cb = lax.rem(my - 1 + s + 2 * N_DEV, N_DEV)
                    recv_cb = lax.rem(my + s + 2 * N_DEV, N_DEV)
                sa = pl.multiple_of(send_ca * CHUNK, CHUNK)
                ra = pl.multiple_of(recv_ca * CHUNK, CHUNK)
                sb = pl.multiple_of(send_cb * CHUNK, CHUNK)
                rb = pl.multiple_of(recv_cb * CHUNK, CHUNK)

                pl.semaphore_wait(credit_a, 1)
                if RING_B:
                    pl.semaphore_wait(credit_b, 1)
                rdma_a = pltpu.make_async_remote_copy(
                    src_ref=out_ref.at[pl.ds(0, HB), pl.ds(sa, CHUNK), :],
                    dst_ref=recv_a,
                    send_sem=send_sem_a,
                    recv_sem=recv_sem_a,
                    device_id=(right,),
                    device_id_type=pl.DeviceIdType.MESH,
                )
                rdma_b = pltpu.make_async_remote_copy(
                    src_ref=out_ref.at[pl.ds(HB, HB), pl.ds(sb, CHUNK), :],
                    dst_ref=recv_b,
                    send_sem=send_sem_b,
                    recv_sem=recv_sem_b,
                    device_id=(left,),
                    device_id_type=pl.DeviceIdType.MESH,
                )
                rdma_a.start()
                if RING_B:
                    rdma_b.start()

                if phase == 0:
                    ld_a = pltpu.make_async_copy(
                        out_ref.at[pl.ds(0, HB), pl.ds(ra, CHUNK), :],
                        acc_a, copy_sem,
                    )
                    ld_b = pltpu.make_async_copy(
                        out_ref.at[pl.ds(HB, HB), pl.ds(rb, CHUNK), :],
                        acc_b, copy_sem,
                    )
                    ld_a.start()
                    ld_b.start()
                    ld_a.wait()
                    ld_b.wait()

                rdma_a.wait()
                if RING_B:
                    rdma_b.wait()

                if phase == 0:
                    acc_a[:, :, :] = acc_a[:, :, :] + recv_a[:, :, :]
                    acc_b[:, :, :] = acc_b[:, :, :] + recv_b[:, :, :]
                    st_a = pltpu.make_async_copy(
                        acc_a, out_ref.at[pl.ds(0, HB), pl.ds(ra, CHUNK), :],
                        store_sem,
                    )
                    st_b = pltpu.make_async_copy(
                        acc_b, out_ref.at[pl.ds(HB, HB), pl.ds(rb, CHUNK), :],
                        store_sem,
                    )
                else:
                    st_a = pltpu.make_async_copy(
                        recv_a, out_ref.at[pl.ds(0, HB), pl.ds(ra, CHUNK), :],
                        store_sem,
                    )
                    st_b = pltpu.make_async_copy(
                        recv_b, out_ref.at[pl.ds(HB, HB), pl.ds(rb, CHUNK), :],
                        store_sem,
                    )
                st_a.start()
                st_a.wait()
                if RING_B:
                    st_b.start()
                    st_b.wait()

                step += 1
                if step < total_steps:
                    pl.semaphore_signal(
                        credit_a, inc=1, device_id=(left,),
                        device_id_type=pl.DeviceIdType.MESH,
                    )
                    if RING_B:
                        pl.semaphore_signal(
                            credit_b, inc=1, device_id=(right,),
                            device_id_type=pl.DeviceIdType.MESH,
                        )

    return pl.pallas_call(
        body,
        out_shape=jax.ShapeDtypeStruct((B, S, C), jnp.float32),
        in_specs=[
            pl.BlockSpec(memory_space=pl.ANY),
            pl.BlockSpec(memory_space=pltpu.VMEM),
            pl.BlockSpec(memory_space=pltpu.VMEM),
        ],
        out_specs=pl.BlockSpec(memory_space=pl.ANY),
        scratch_shapes=[
            pltpu.VMEM((B, CC + HALO, C), jnp.float32),
            pltpu.VMEM((B, CC, C), jnp.float32),
            pltpu.VMEM((HB, CHUNK, C), jnp.float32),
            pltpu.VMEM((HB, CHUNK, C), jnp.float32),
            pltpu.VMEM((HB, CHUNK, C), jnp.float32),
            pltpu.VMEM((HB, CHUNK, C), jnp.float32),
            pltpu.SemaphoreType.DMA,
            pltpu.SemaphoreType.DMA,
            pltpu.SemaphoreType.DMA,
            pltpu.SemaphoreType.DMA,
            pltpu.SemaphoreType.DMA,
            pltpu.SemaphoreType.DMA,
            pltpu.SemaphoreType.REGULAR,
            pltpu.SemaphoreType.REGULAR,
        ],
        compiler_params=pltpu.CompilerParams(collective_id=0),
    )(x, k, Wp)


# device time: 434900 ns/iter; 1.9435x vs baseline; 1.2075x over previous
import jax
import jax.numpy as jnp
from jax import lax
from jax.experimental import pallas as pl
from jax.experimental.pallas import tpu as pltpu

N_DEV = 8
B, S, C = 4, 2048, 1024
TAPS = 4
CHUNK = S // N_DEV
HB = B // 2
HALO = 8
OVERLAP_COMPUTE = True


def kernel(x, k, Wp):
    def body(
        x_ref, k_ref, w_ref, out_ref,
        xbuf, obuf, acc_a, acc_b, recv_a, recv_b,
        copy_sem, xcp_sem, hcp_sem, store_sem,
        send_sem_a, recv_sem_a, send_sem_b, recv_sem_b,
        credit_a, credit_b,
    ):
        my = lax.axis_index("i")
        left = lax.rem(my + N_DEV - 1, N_DEV)
        right = lax.rem(my + 1, N_DEV)

        def compute_half(c_idx, boff):
            base = pl.multiple_of(c_idx * CHUNK, CHUNK)
            hstart = pl.multiple_of(jnp.maximum(base - HALO, 0), HALO)
            cp = pltpu.make_async_copy(
                x_ref.at[pl.ds(boff, HB), pl.ds(base, CHUNK), :],
                xbuf.at[:, pl.ds(HALO, CHUNK), :],
                xcp_sem,
            )
            hp = pltpu.make_async_copy(
                x_ref.at[pl.ds(boff, HB), pl.ds(hstart, HALO), :],
                xbuf.at[:, pl.ds(0, HALO), :],
                hcp_sem,
            )
            cp.start()
            hp.start()
            cp.wait()
            hp.wait()
            halo_mask = jnp.where(c_idx == 0, 0.0, 1.0)
            xbuf[:, :HALO, :] = xbuf[:, :HALO, :] * halo_mask
            acc = xbuf[:, pl.ds(HALO, CHUNK), :] * k_ref[TAPS - 1, :][None, None, :]
            for d in range(1, TAPS):
                tap = k_ref[TAPS - 1 - d, :][None, None, :]
                acc = acc + xbuf[:, pl.ds(HALO - d, CHUNK), :] * tap
            a = acc * jax.nn.sigmoid(acc)
            for b in range(HB):
                obuf[b, :, :] = jnp.dot(
                    a[b], w_ref[:, :], preferred_element_type=jnp.float32
                )
            st = pltpu.make_async_copy(
                obuf,
                out_ref.at[pl.ds(boff, HB), pl.ds(base, CHUNK), :],
                store_sem,
            )
            st.start()
            st.wait()

        barrier = pltpu.get_barrier_semaphore()
        for nbr in (left, right):
            pl.semaphore_signal(
                barrier, inc=1, device_id=(nbr,),
                device_id_type=pl.DeviceIdType.MESH,
            )
        pl.semaphore_wait(barrier, 2)

        pl.semaphore_signal(
            credit_a, inc=1, device_id=(left,),
            device_id_type=pl.DeviceIdType.MESH,
        )
        pl.semaphore_signal(
            credit_b, inc=1, device_id=(right,),
            device_id_type=pl.DeviceIdType.MESH,
        )

        compute_half(my, 0)
        compute_half(my, HB)
        if not OVERLAP_COMPUTE:
            for j in range(1, N_DEV):
                compute_half(lax.rem(my + j + N_DEV, N_DEV), 0)
                compute_half(lax.rem(my + j + N_DEV, N_DEV), HB)

        total_steps = 2 * (N_DEV - 1)
        step = 0
        for phase in range(2):
            for s in range(N_DEV - 1):
                if phase == 0:
                    send_ca = lax.rem(my - s + 2 * N_DEV, N_DEV)
                    recv_ca = lax.rem(my - s - 1 + 2 * N_DEV, N_DEV)
                    send_cb = lax.rem(my + s + 2 * N_DEV, N_DEV)
                    recv_cb = lax.rem(my + s + 1 + 2 * N_DEV, N_DEV)
                else:
                    send_ca = lax.rem(my + 1 - s + 2 * N_DEV, N_DEV)
                    recv_ca = lax.rem(my - s + 2 * N_DEV, N_DEV)
                    send_cb = lax.rem(my - 1 + s + 2 * N_DEV, N_DEV)
                    recv_cb = lax.rem(my + s + 2 * N_DEV, N_DEV)
                sa = pl.multiple_of(send_ca * CHUNK, CHUNK)
                ra = pl.multiple_of(recv_ca * CHUNK, CHUNK)
                sb = pl.multiple_of(send_cb * CHUNK, CHUNK)
                rb = pl.multiple_of(recv_cb * CHUNK, CHUNK)

                pl.semaphore_wait(credit_a, 1)
                pl.semaphore_wait(credit_b, 1)
                rdma_a = pltpu.make_async_remote_copy(
                    src_ref=out_ref.at[pl.ds(0, HB), pl.ds(sa, CHUNK), :],
                    dst_ref=recv_a,
                    send_sem=send_sem_a,
                    recv_sem=recv_sem_a,
                    device_id=(right,),
                    device_id_type=pl.DeviceIdType.MESH,
                )
                rdma_b = pltpu.make_async_remote_copy(
                    src_ref=out_ref.at[pl.ds(HB, HB), pl.ds(sb, CHUNK), :],
                    dst_ref=recv_b,
                    send_sem=send_sem_b,
                    recv_sem=recv_sem_b,
                    device_id=(left,),
                    device_id_type=pl.DeviceIdType.MESH,
                )
                rdma_a.start()
                rdma_b.start()

                if phase == 0:
                    if OVERLAP_COMPUTE:
                        compute_half(recv_ca, 0)
                        compute_half(recv_cb, HB)
                    ld_a = pltpu.make_async_copy(
                        out_ref.at[pl.ds(0, HB), pl.ds(ra, CHUNK), :],
                        acc_a, copy_sem,
                    )
                    ld_b = pltpu.make_async_copy(
                        out_ref.at[pl.ds(HB, HB), pl.ds(rb, CHUNK), :],
                        acc_b, copy_sem,
                    )
                    ld_a.start()
                    ld_b.start()
                    ld_a.wait()
                    ld_b.wait()

                rdma_a.wait()
                rdma_b.wait()

                if phase == 0:
                    acc_a[:, :, :] = acc_a[:, :, :] + recv_a[:, :, :]
                    acc_b[:, :, :] = acc_b[:, :, :] + recv_b[:, :, :]
                    st_a = pltpu.make_async_copy(
                        acc_a, out_ref.at[pl.ds(0, HB), pl.ds(ra, CHUNK), :],
                        store_sem,
                    )
                    st_b = pltpu.make_async_copy(
                        acc_b, out_ref.at[pl.ds(HB, HB), pl.ds(rb, CHUNK), :],
                        store_sem,
                    )
                else:
                    st_a = pltpu.make_async_copy(
                        recv_a, out_ref.at[pl.ds(0, HB), pl.ds(ra, CHUNK), :],
                        store_sem,
                    )
                    st_b = pltpu.make_async_copy(
                        recv_b, out_ref.at[pl.ds(HB, HB), pl.ds(rb, CHUNK), :],
                        store_sem,
                    )
                st_a.start()
                st_b.start()
                st_a.wait()
                st_b.wait()

                step += 1
                if step < total_steps:
                    pl.semaphore_signal(
                        credit_a, inc=1, device_id=(left,),
                        device_id_type=pl.DeviceIdType.MESH,
                    )
                    pl.semaphore_signal(
                        credit_b, inc=1, device_id=(right,),
                        device_id_type=pl.DeviceIdType.MESH,
                    )

    return pl.pallas_call(
        body,
        out_shape=jax.ShapeDtypeStruct((B, S, C), jnp.float32),
        in_specs=[
            pl.BlockSpec(memory_space=pl.ANY),
            pl.BlockSpec(memory_space=pltpu.VMEM),
            pl.BlockSpec(memory_space=pltpu.VMEM),
        ],
        out_specs=pl.BlockSpec(memory_space=pl.ANY),
        scratch_shapes=[
            pltpu.VMEM((HB, CHUNK + HALO, C), jnp.float32),
            pltpu.VMEM((HB, CHUNK, C), jnp.float32),
            pltpu.VMEM((HB, CHUNK, C), jnp.float32),
            pltpu.VMEM((HB, CHUNK, C), jnp.float32),
            pltpu.VMEM((HB, CHUNK, C), jnp.float32),
            pltpu.VMEM((HB, CHUNK, C), jnp.float32),
            pltpu.SemaphoreType.DMA,
            pltpu.SemaphoreType.DMA,
            pltpu.SemaphoreType.DMA,
            pltpu.SemaphoreType.DMA,
            pltpu.SemaphoreType.DMA,
            pltpu.SemaphoreType.DMA,
            pltpu.SemaphoreType.DMA,
            pltpu.SemaphoreType.DMA,
            pltpu.SemaphoreType.REGULAR,
            pltpu.SemaphoreType.REGULAR,
        ],
        compiler_params=pltpu.CompilerParams(collective_id=0),
    )(x, k, Wp)


# device time: 331031 ns/iter; 2.5533x vs baseline; 1.3138x over previous
import jax
import jax.numpy as jnp
from jax import lax
from jax.experimental import pallas as pl
from jax.experimental.pallas import tpu as pltpu

N_DEV = 8
B, S, C = 4, 2048, 1024
TAPS = 4
CHUNK = S // N_DEV
HB = B // 2
HALO = 8


def kernel(x, k, Wp):
    def body(
        x_ref, k_ref, w_ref, out_ref,
        xbuf, pbuf_a, pbuf_b, recv_a, recv_b, agr_a, agr_b, agseed_a, agseed_b,
        xcp_sem, hcp_sem, store_sem,
        send_sem_a, recv_sem_a, send_sem_b, recv_sem_b,
        credit_a, credit_b,
    ):
        my = lax.axis_index("i")
        left = lax.rem(my + N_DEV - 1, N_DEV)
        right = lax.rem(my + 1, N_DEV)

        def compute_half(c_idx, boff, dst):
            base = pl.multiple_of(c_idx * CHUNK, CHUNK)
            hstart = pl.multiple_of(jnp.maximum(base - HALO, 0), HALO)
            cp = pltpu.make_async_copy(
                x_ref.at[pl.ds(boff, HB), pl.ds(base, CHUNK), :],
                xbuf.at[:, pl.ds(HALO, CHUNK), :],
                xcp_sem,
            )
            hp = pltpu.make_async_copy(
                x_ref.at[pl.ds(boff, HB), pl.ds(hstart, HALO), :],
                xbuf.at[:, pl.ds(0, HALO), :],
                hcp_sem,
            )
            cp.start()
            hp.start()
            cp.wait()
            hp.wait()
            halo_mask = jnp.where(c_idx == 0, 0.0, 1.0)
            xbuf[:, :HALO, :] = xbuf[:, :HALO, :] * halo_mask
            acc = xbuf[:, pl.ds(HALO, CHUNK), :] * k_ref[TAPS - 1, :][None, None, :]
            for d in range(1, TAPS):
                tap = k_ref[TAPS - 1 - d, :][None, None, :]
                acc = acc + xbuf[:, pl.ds(HALO - d, CHUNK), :] * tap
            a = acc * jax.nn.sigmoid(acc)
            for b in range(HB):
                dst[b, :, :] = jnp.dot(
                    a[b], w_ref[:, :], preferred_element_type=jnp.float32
                )

        barrier = pltpu.get_barrier_semaphore()
        for nbr in (left, right):
            pl.semaphore_signal(
                barrier, inc=1, device_id=(nbr,),
                device_id_type=pl.DeviceIdType.MESH,
            )
        pl.semaphore_wait(barrier, 2)

        pl.semaphore_signal(
            credit_a, inc=1, device_id=(left,),
            device_id_type=pl.DeviceIdType.MESH,
        )
        pl.semaphore_signal(
            credit_b, inc=1, device_id=(right,),
            device_id_type=pl.DeviceIdType.MESH,
        )

        compute_half(my, 0, pbuf_a.at[0])
        compute_half(my, HB, pbuf_b.at[0])

        for s in range(N_DEV - 1):
            recv_ca = lax.rem(my - s - 1 + 2 * N_DEV, N_DEV)
            recv_cb = lax.rem(my + s + 1 + 2 * N_DEV, N_DEV)
            cur = s % 2
            nxt = (s + 1) % 2

            pl.semaphore_wait(credit_a, 1)
            pl.semaphore_wait(credit_b, 1)
            rdma_a = pltpu.make_async_remote_copy(
                src_ref=pbuf_a.at[cur],
                dst_ref=recv_a,
                send_sem=send_sem_a,
                recv_sem=recv_sem_a,
                device_id=(right,),
                device_id_type=pl.DeviceIdType.MESH,
            )
            rdma_b = pltpu.make_async_remote_copy(
                src_ref=pbuf_b.at[cur],
                dst_ref=recv_b,
                send_sem=send_sem_b,
                recv_sem=recv_sem_b,
                device_id=(left,),
                device_id_type=pl.DeviceIdType.MESH,
            )
            rdma_a.start()
            rdma_b.start()

            compute_half(recv_ca, 0, pbuf_a.at[nxt])
            compute_half(recv_cb, HB, pbuf_b.at[nxt])

            rdma_a.wait()
            rdma_b.wait()

            pbuf_a[nxt, :, :, :] = pbuf_a[nxt, :, :, :] + recv_a[:, :, :]
            pbuf_b[nxt, :, :, :] = pbuf_b[nxt, :, :, :] + recv_b[:, :, :]

            if s < N_DEV - 2:
                pl.semaphore_signal(
                    credit_a, inc=1, device_id=(left,),
                    device_id_type=pl.DeviceIdType.MESH,
                )
                pl.semaphore_signal(
                    credit_b, inc=1, device_id=(right,),
                    device_id_type=pl.DeviceIdType.MESH,
                )

        own_a = pl.multiple_of(lax.rem(my + 1, N_DEV) * CHUNK, CHUNK)
        own_b = pl.multiple_of(lax.rem(my - 1 + N_DEV, N_DEV) * CHUNK, CHUNK)
        fin_a = pltpu.make_async_copy(
            pbuf_a.at[1], out_ref.at[pl.ds(0, HB), pl.ds(own_a, CHUNK), :],
            store_sem,
        )
        fin_b = pltpu.make_async_copy(
            pbuf_b.at[1], out_ref.at[pl.ds(HB, HB), pl.ds(own_b, CHUNK), :],
            store_sem,
        )
        fin_a.start()
        fin_b.start()
        agseed_a[:, :, :] = pbuf_a[1, :, :, :].astype(jnp.bfloat16)
        agseed_b[:, :, :] = pbuf_b[1, :, :, :].astype(jnp.bfloat16)
        fin_a.wait()
        fin_b.wait()

        pl.semaphore_signal(
            credit_a, inc=2, device_id=(left,),
            device_id_type=pl.DeviceIdType.MESH,
        )
        pl.semaphore_signal(
            credit_b, inc=2, device_id=(right,),
            device_id_type=pl.DeviceIdType.MESH,
        )

        for s in range(N_DEV - 1):
            ra = pl.multiple_of(
                lax.rem(my - s + 2 * N_DEV, N_DEV) * CHUNK, CHUNK
            )
            rb = pl.multiple_of(
                lax.rem(my + s + 2 * N_DEV, N_DEV) * CHUNK, CHUNK
            )
            cur = s % 2
            prv = (s - 1) % 2

            pl.semaphore_wait(credit_a, 1)
            pl.semaphore_wait(credit_b, 1)
            rdma_a = pltpu.make_async_remote_copy(
                src_ref=agseed_a if s == 0 else agr_a.at[prv],
                dst_ref=agr_a.at[cur],
                send_sem=send_sem_a,
                recv_sem=recv_sem_a,
                device_id=(right,),
                device_id_type=pl.DeviceIdType.MESH,
            )
            rdma_b = pltpu.make_async_remote_copy(
                src_ref=agseed_b if s == 0 else agr_b.at[prv],
                dst_ref=agr_b.at[cur],
                send_sem=send_sem_b,
                recv_sem=recv_sem_b,
                device_id=(left,),
                device_id_type=pl.DeviceIdType.MESH,
            )
            rdma_a.start()
            rdma_b.start()
            rdma_a.wait()
            rdma_b.wait()

            recv_a[:, :, :] = agr_a[cur, :, :, :].astype(jnp.float32)
            recv_b[:, :, :] = agr_b[cur, :, :, :].astype(jnp.float32)
            st_a = pltpu.make_async_copy(
                recv_a, out_ref.at[pl.ds(0, HB), pl.ds(ra, CHUNK), :],
                store_sem,
            )
            st_b = pltpu.make_async_copy(
                recv_b, out_ref.at[pl.ds(HB, HB), pl.ds(rb, CHUNK), :],
                store_sem,
            )
            st_a.start()
            st_b.start()

            if 1 <= s <= 5:
                pl.semaphore_signal(
                    credit_a, inc=1, device_id=(left,),
                    device_id_type=pl.DeviceIdType.MESH,
                )
                pl.semaphore_signal(
                    credit_b, inc=1, device_id=(right,),
                    device_id_type=pl.DeviceIdType.MESH,
                )

            st_a.wait()
            st_b.wait()

    return pl.pallas_call(
        body,
        out_shape=jax.ShapeDtypeStruct((B, S, C), jnp.float32),
        in_specs=[
            pl.BlockSpec(memory_space=pl.ANY),
            pl.BlockSpec(memory_space=pltpu.VMEM),
            pl.BlockSpec(memory_space=pltpu.VMEM),
        ],
        out_specs=pl.BlockSpec(memory_space=pl.ANY),
        scratch_shapes=[
            pltpu.VMEM((HB, CHUNK + HALO, C), jnp.float32),
            pltpu.VMEM((2, HB, CHUNK, C), jnp.float32),
            pltpu.VMEM((2, HB, CHUNK, C), jnp.float32),
            pltpu.VMEM((HB, CHUNK, C), jnp.float32),
            pltpu.VMEM((HB, CHUNK, C), jnp.float32),
            pltpu.VMEM((2, HB, CHUNK, C), jnp.bfloat16),
            pltpu.VMEM((2, HB, CHUNK, C), jnp.bfloat16),
            pltpu.VMEM((HB, CHUNK, C), jnp.bfloat16),
            pltpu.VMEM((HB, CHUNK, C), jnp.bfloat16),
            pltpu.SemaphoreType.DMA,
            pltpu.SemaphoreType.DMA,
            pltpu.SemaphoreType.DMA,
            pltpu.SemaphoreType.DMA,
            pltpu.SemaphoreType.DMA,
            pltpu.SemaphoreType.DMA,
            pltpu.SemaphoreType.DMA,
            pltpu.SemaphoreType.REGULAR,
            pltpu.SemaphoreType.REGULAR,
        ],
        compiler_params=pltpu.CompilerParams(
            collective_id=0, vmem_limit_bytes=56 * 1024 * 1024
        ),
    )(x, k, Wp)


# device time: 254560 ns/iter; 3.3203x vs baseline; 1.3004x over previous
import jax
import jax.numpy as jnp
from jax import lax
from jax.experimental import pallas as pl
from jax.experimental.pallas import tpu as pltpu

N_DEV = 8
B, S, C = 4, 2048, 1024
TAPS = 4
CHUNK = S // N_DEV
HB = B // 2
HALO = 8


def kernel(x, k, Wp):
    def body(
        x_ref, k_ref, w_ref, out_ref,
        xbuf, wbuf, pbuf_a, pbuf_b, rs_recv_a, rs_recv_b, sbuf_a, sbuf_b,
        agr_a, agr_b, stg_a, stg_b,
        xcp_sem, hcp_sem, store_sem,
        send_sem_a, recv_sem_a, send_sem_b, recv_sem_b,
        credit_a, credit_b,
    ):
        my = lax.axis_index("i")
        left = lax.rem(my + N_DEV - 1, N_DEV)
        right = lax.rem(my + 1, N_DEV)

        def compute_half(c_idx, boff, dst):
            base = pl.multiple_of(c_idx * CHUNK, CHUNK)
            hstart = pl.multiple_of(jnp.maximum(base - HALO, 0), HALO)
            cp = pltpu.make_async_copy(
                x_ref.at[pl.ds(boff, HB), pl.ds(base, CHUNK), :],
                xbuf.at[:, pl.ds(HALO, CHUNK), :],
                xcp_sem,
            )
            hp = pltpu.make_async_copy(
                x_ref.at[pl.ds(boff, HB), pl.ds(hstart, HALO), :],
                xbuf.at[:, pl.ds(0, HALO), :],
                hcp_sem,
            )
            cp.start()
            hp.start()
            cp.wait()
            hp.wait()
            halo_mask = jnp.where(c_idx == 0, 0.0, 1.0)
            xbuf[:, :HALO, :] = xbuf[:, :HALO, :] * halo_mask
            acc = xbuf[:, pl.ds(HALO, CHUNK), :] * k_ref[TAPS - 1, :][None, None, :]
            for d in range(1, TAPS):
                tap = k_ref[TAPS - 1 - d, :][None, None, :]
                acc = acc + xbuf[:, pl.ds(HALO - d, CHUNK), :] * tap
            a = (acc * jax.nn.sigmoid(acc)).astype(jnp.bfloat16)
            for b in range(HB):
                dst[b, :, :] = jnp.dot(
                    a[b], wbuf[:, :], preferred_element_type=jnp.float32
                )

        barrier = pltpu.get_barrier_semaphore()
        for nbr in (left, right):
            pl.semaphore_signal(
                barrier, inc=1, device_id=(nbr,),
                device_id_type=pl.DeviceIdType.MESH,
            )
        pl.semaphore_wait(barrier, 2)

        pl.semaphore_signal(
            credit_a, inc=1, device_id=(left,),
            device_id_type=pl.DeviceIdType.MESH,
        )
        pl.semaphore_signal(
            credit_b, inc=1, device_id=(right,),
            device_id_type=pl.DeviceIdType.MESH,
        )

        wbuf[:, :] = w_ref[:, :].astype(jnp.bfloat16)

        compute_half(my, 0, pbuf_a.at[0])
        compute_half(my, HB, pbuf_b.at[0])
        sbuf_a[:, :, :] = pbuf_a[0, :, :, :].astype(jnp.bfloat16)
        sbuf_b[:, :, :] = pbuf_b[0, :, :, :].astype(jnp.bfloat16)

        for s in range(N_DEV - 1):
            recv_ca = lax.rem(my - s - 1 + 2 * N_DEV, N_DEV)
            recv_cb = lax.rem(my + s + 1 + 2 * N_DEV, N_DEV)
            cur = s % 2
            nxt = (s + 1) % 2

            pl.semaphore_wait(credit_a, 1)
            pl.semaphore_wait(credit_b, 1)
            rdma_a = pltpu.make_async_remote_copy(
                src_ref=sbuf_a,
                dst_ref=rs_recv_a,
                send_sem=send_sem_a,
                recv_sem=recv_sem_a,
                device_id=(right,),
                device_id_type=pl.DeviceIdType.MESH,
            )
            rdma_b = pltpu.make_async_remote_copy(
                src_ref=sbuf_b,
                dst_ref=rs_recv_b,
                send_sem=send_sem_b,
                recv_sem=recv_sem_b,
                device_id=(left,),
                device_id_type=pl.DeviceIdType.MESH,
            )
            rdma_a.start()
            rdma_b.start()

            compute_half(recv_ca, 0, pbuf_a.at[nxt])
            compute_half(recv_cb, HB, pbuf_b.at[nxt])

            rdma_a.wait()
            rdma_b.wait()

            pbuf_a[nxt, :, :, :] = pbuf_a[nxt, :, :, :] + rs_recv_a[
                :, :, :
            ].astype(jnp.float32)
            pbuf_b[nxt, :, :, :] = pbuf_b[nxt, :, :, :] + rs_recv_b[
                :, :, :
            ].astype(jnp.float32)
            sbuf_a[:, :, :] = pbuf_a[nxt, :, :, :].astype(jnp.bfloat16)
            sbuf_b[:, :, :] = pbuf_b[nxt, :, :, :].astype(jnp.bfloat16)

            if s < N_DEV - 2:
                pl.semaphore_signal(
                    credit_a, inc=1, device_id=(left,),
                    device_id_type=pl.DeviceIdType.MESH,
                )
                pl.semaphore_signal(
                    credit_b, inc=1, device_id=(right,),
                    device_id_type=pl.DeviceIdType.MESH,
                )

        own_a = pl.multiple_of(lax.rem(my + 1, N_DEV) * CHUNK, CHUNK)
        own_b = pl.multiple_of(lax.rem(my - 1 + N_DEV, N_DEV) * CHUNK, CHUNK)
        fin_a = pltpu.make_async_copy(
            pbuf_a.at[1], out_ref.at[pl.ds(0, HB), pl.ds(own_a, CHUNK), :],
            store_sem,
        )
        fin_b = pltpu.make_async_copy(
            pbuf_b.at[1], out_ref.at[pl.ds(HB, HB), pl.ds(own_b, CHUNK), :],
            store_sem,
        )
        fin_a.start()
        fin_b.start()
        fin_a.wait()
        fin_b.wait()

        pl.semaphore_signal(
            credit_a, inc=2, device_id=(left,),
            device_id_type=pl.DeviceIdType.MESH,
        )
        pl.semaphore_signal(
            credit_b, inc=2, device_id=(right,),
            device_id_type=pl.DeviceIdType.MESH,
        )

        for s in range(N_DEV - 1):
            ra = pl.multiple_of(
                lax.rem(my - s + 2 * N_DEV, N_DEV) * CHUNK, CHUNK
            )
            rb = pl.multiple_of(
                lax.rem(my + s + 2 * N_DEV, N_DEV) * CHUNK, CHUNK
            )
            cur = s % 2
            prv = (s - 1) % 2

            pl.semaphore_wait(credit_a, 1)
            pl.semaphore_wait(credit_b, 1)
            rdma_a = pltpu.make_async_remote_copy(
                src_ref=sbuf_a if s == 0 else agr_a.at[prv],
                dst_ref=agr_a.at[cur],
                send_sem=send_sem_a,
                recv_sem=recv_sem_a,
                device_id=(right,),
                device_id_type=pl.DeviceIdType.MESH,
            )
            rdma_b = pltpu.make_async_remote_copy(
                src_ref=sbuf_b if s == 0 else agr_b.at[prv],
                dst_ref=agr_b.at[cur],
                send_sem=send_sem_b,
                recv_sem=recv_sem_b,
                device_id=(left,),
                device_id_type=pl.DeviceIdType.MESH,
            )
            rdma_a.start()
            rdma_b.start()
            rdma_a.wait()
            rdma_b.wait()

            stg_a[:, :, :] = agr_a[cur, :, :, :].astype(jnp.float32)
            stg_b[:, :, :] = agr_b[cur, :, :, :].astype(jnp.float32)
            st_a = pltpu.make_async_copy(
                stg_a, out_ref.at[pl.ds(0, HB), pl.ds(ra, CHUNK), :],
                store_sem,
            )
            st_b = pltpu.make_async_copy(
                stg_b, out_ref.at[pl.ds(HB, HB), pl.ds(rb, CHUNK), :],
                store_sem,
            )
            st_a.start()
            st_b.start()

            if 1 <= s <= 5:
                pl.semaphore_signal(
                    credit_a, inc=1, device_id=(left,),
                    device_id_type=pl.DeviceIdType.MESH,
                )
                pl.semaphore_signal(
                    credit_b, inc=1, device_id=(right,),
                    device_id_type=pl.DeviceIdType.MESH,
                )

            st_a.wait()
            st_b.wait()

    return pl.pallas_call(
        body,
        out_shape=jax.ShapeDtypeStruct((B, S, C), jnp.float32),
        in_specs=[
            pl.BlockSpec(memory_space=pl.ANY),
            pl.BlockSpec(memory_space=pltpu.VMEM),
            pl.BlockSpec(memory_space=pltpu.VMEM),
        ],
        out_specs=pl.BlockSpec(memory_space=pl.ANY),
        scratch_shapes=[
            pltpu.VMEM((HB, CHUNK + HALO, C), jnp.float32),
            pltpu.VMEM((C, C), jnp.bfloat16),
            pltpu.VMEM((2, HB, CHUNK, C), jnp.float32),
            pltpu.VMEM((2, HB, CHUNK, C), jnp.float32),
            pltpu.VMEM((HB, CHUNK, C), jnp.bfloat16),
            pltpu.VMEM((HB, CHUNK, C), jnp.bfloat16),
            pltpu.VMEM((HB, CHUNK, C), jnp.bfloat16),
            pltpu.VMEM((HB, CHUNK, C), jnp.bfloat16),
            pltpu.VMEM((2, HB, CHUNK, C), jnp.bfloat16),
            pltpu.VMEM((2, HB, CHUNK, C), jnp.bfloat16),
            pltpu.VMEM((HB, CHUNK, C), jnp.float32),
            pltpu.VMEM((HB, CHUNK, C), jnp.float32),
            pltpu.SemaphoreType.DMA,
            pltpu.SemaphoreType.DMA,
            pltpu.SemaphoreType.DMA,
            pltpu.SemaphoreType.DMA,
            pltpu.SemaphoreType.DMA,
            pltpu.SemaphoreType.DMA,
            pltpu.SemaphoreType.DMA,
            pltpu.SemaphoreType.REGULAR,
            pltpu.SemaphoreType.REGULAR,
        ],
        compiler_params=pltpu.CompilerParams(
            collective_id=0, vmem_limit_bytes=56 * 1024 * 1024
        ),
    )(x, k, Wp)


# device time: 236615 ns/iter; 3.5721x vs baseline; 1.0758x over previous
import jax
import jax.numpy as jnp
from jax import lax
from jax.experimental import pallas as pl
from jax.experimental.pallas import tpu as pltpu

N_DEV = 8
B, S, C = 4, 2048, 1024
TAPS = 4
CHUNK = S // N_DEV
HB = B // 2
HALO = 8


def kernel(x, k, Wp):
    def body(
        x_ref, k_ref, w_ref, out_ref,
        xbuf, wbuf, pbuf_a, pbuf_b, rs_recv_a, rs_recv_b, sbuf_a, sbuf_b,
        agr_a, agr_b, stg_a, stg_b,
        xcp_sem, hcp_sem, store_sem,
        send_sem_a, recv_sem_a, send_sem_b, recv_sem_b,
        credit_a, credit_b,
    ):
        def gray(i):
            return jnp.where(i < 4, i, 11 - i)

        mesh_id = lax.axis_index("i")
        my = gray(mesh_id)
        left = gray(lax.rem(my + N_DEV - 1, N_DEV))
        right = gray(lax.rem(my + 1, N_DEV))

        def compute_half(c_idx, boff, dst):
            base = pl.multiple_of(c_idx * CHUNK, CHUNK)
            hstart = pl.multiple_of(jnp.maximum(base - HALO, 0), HALO)
            cp = pltpu.make_async_copy(
                x_ref.at[pl.ds(boff, HB), pl.ds(base, CHUNK), :],
                xbuf.at[:, pl.ds(HALO, CHUNK), :],
                xcp_sem,
            )
            hp = pltpu.make_async_copy(
                x_ref.at[pl.ds(boff, HB), pl.ds(hstart, HALO), :],
                xbuf.at[:, pl.ds(0, HALO), :],
                hcp_sem,
            )
            cp.start()
            hp.start()
            cp.wait()
            hp.wait()
            halo_mask = jnp.where(c_idx == 0, 0.0, 1.0)
            xbuf[:, :HALO, :] = xbuf[:, :HALO, :] * halo_mask
            acc = xbuf[:, pl.ds(HALO, CHUNK), :] * k_ref[TAPS - 1, :][None, None, :]
            for d in range(1, TAPS):
                tap = k_ref[TAPS - 1 - d, :][None, None, :]
                acc = acc + xbuf[:, pl.ds(HALO - d, CHUNK), :] * tap
            a = (acc * jax.nn.sigmoid(acc)).astype(jnp.bfloat16)
            for b in range(HB):
                dst[b, :, :] = jnp.dot(
                    a[b], wbuf[:, :], preferred_element_type=jnp.float32
                )

        barrier = pltpu.get_barrier_semaphore()
        for nbr in (left, right):
            pl.semaphore_signal(
                barrier, inc=1, device_id=(nbr,),
                device_id_type=pl.DeviceIdType.MESH,
            )
        pl.semaphore_wait(barrier, 2)

        pl.semaphore_signal(
            credit_a, inc=1, device_id=(left,),
            device_id_type=pl.DeviceIdType.MESH,
        )
        pl.semaphore_signal(
            credit_b, inc=1, device_id=(right,),
            device_id_type=pl.DeviceIdType.MESH,
        )

        wbuf[:, :] = w_ref[:, :].astype(jnp.bfloat16)

        compute_half(my, 0, pbuf_a.at[0])
        compute_half(my, HB, pbuf_b.at[0])
        sbuf_a[:, :, :] = pbuf_a[0, :, :, :].astype(jnp.bfloat16)
        sbuf_b[:, :, :] = pbuf_b[0, :, :, :].astype(jnp.bfloat16)

        for s in range(N_DEV - 1):
            recv_ca = lax.rem(my - s - 1 + 2 * N_DEV, N_DEV)
            recv_cb = lax.rem(my + s + 1 + 2 * N_DEV, N_DEV)
            cur = s % 2
            nxt = (s + 1) % 2

            pl.semaphore_wait(credit_a, 1)
            pl.semaphore_wait(credit_b, 1)
            rdma_a = pltpu.make_async_remote_copy(
                src_ref=sbuf_a,
                dst_ref=rs_recv_a,
                send_sem=send_sem_a,
                recv_sem=recv_sem_a,
                device_id=(right,),
                device_id_type=pl.DeviceIdType.MESH,
            )
            rdma_b = pltpu.make_async_remote_copy(
                src_ref=sbuf_b,
                dst_ref=rs_recv_b,
                send_sem=send_sem_b,
                recv_sem=recv_sem_b,
                device_id=(left,),
                device_id_type=pl.DeviceIdType.MESH,
            )
            rdma_a.start()
            rdma_b.start()

            compute_half(recv_ca, 0, pbuf_a.at[nxt])
            compute_half(recv_cb, HB, pbuf_b.at[nxt])

            rdma_a.wait()
            rdma_b.wait()

            pbuf_a[nxt, :, :, :] = pbuf_a[nxt, :, :, :] + rs_recv_a[
                :, :, :
            ].astype(jnp.float32)
            pbuf_b[nxt, :, :, :] = pbuf_b[nxt, :, :, :] + rs_recv_b[
                :, :, :
            ].astype(jnp.float32)
            sbuf_a[:, :, :] = pbuf_a[nxt, :, :, :].astype(jnp.bfloat16)
            sbuf_b[:, :, :] = pbuf_b[nxt, :, :, :].astype(jnp.bfloat16)

            if s < N_DEV - 2:
                pl.semaphore_signal(
                    credit_a, inc=1, device_id=(left,),
                    device_id_type=pl.DeviceIdType.MESH,
                )
                pl.semaphore_signal(
                    credit_b, inc=1, device_id=(right,),
                    device_id_type=pl.DeviceIdType.MESH,
                )

        own_a = pl.multiple_of(lax.rem(my + 1, N_DEV) * CHUNK, CHUNK)
        own_b = pl.multiple_of(lax.rem(my - 1 + N_DEV, N_DEV) * CHUNK, CHUNK)
        fin_a = pltpu.make_async_copy(
            pbuf_a.at[1], out_ref.at[pl.ds(0, HB), pl.ds(own_a, CHUNK), :],
            store_sem,
        )
        fin_b = pltpu.make_async_copy(
            pbuf_b.at[1], out_ref.at[pl.ds(HB, HB), pl.ds(own_b, CHUNK), :],
            store_sem,
        )
        fin_a.start()
        fin_b.start()
        fin_a.wait()
        fin_b.wait()

        pl.semaphore_signal(
            credit_a, inc=2, device_id=(left,),
            device_id_type=pl.DeviceIdType.MESH,
        )
        pl.semaphore_signal(
            credit_b, inc=2, device_id=(right,),
            device_id_type=pl.DeviceIdType.MESH,
        )

        def ag_store(s):
            slot = s % 2
            ra = pl.multiple_of(
                lax.rem(my - s + 2 * N_DEV, N_DEV) * CHUNK, CHUNK
            )
            rb = pl.multiple_of(
                lax.rem(my + s + 2 * N_DEV, N_DEV) * CHUNK, CHUNK
            )
            stg_a[:, :, :] = agr_a[slot, :, :, :].astype(jnp.float32)
            stg_b[:, :, :] = agr_b[slot, :, :, :].astype(jnp.float32)
            st_a = pltpu.make_async_copy(
                stg_a, out_ref.at[pl.ds(0, HB), pl.ds(ra, CHUNK), :],
                store_sem,
            )
            st_b = pltpu.make_async_copy(
                stg_b, out_ref.at[pl.ds(HB, HB), pl.ds(rb, CHUNK), :],
                store_sem,
            )
            st_a.start()
            st_b.start()
            st_a.wait()
            st_b.wait()

        for s in range(N_DEV - 1):
            cur = s % 2
            prv = (s - 1) % 2

            pl.semaphore_wait(credit_a, 1)
            pl.semaphore_wait(credit_b, 1)
            rdma_a = pltpu.make_async_remote_copy(
                src_ref=sbuf_a if s == 0 else agr_a.at[prv],
                dst_ref=agr_a.at[cur],
                send_sem=send_sem_a,
                recv_sem=recv_sem_a,
                device_id=(right,),
                device_id_type=pl.DeviceIdType.MESH,
            )
            rdma_b = pltpu.make_async_remote_copy(
                src_ref=sbuf_b if s == 0 else agr_b.at[prv],
                dst_ref=agr_b.at[cur],
                send_sem=send_sem_b,
                recv_sem=recv_sem_b,
                device_id=(left,),
                device_id_type=pl.DeviceIdType.MESH,
            )
            rdma_a.start()
            rdma_b.start()

            if s >= 1:
                ag_store(s - 1)

            rdma_a.wait()
            rdma_b.wait()

            if 1 <= s <= 5:
                pl.semaphore_signal(
                    credit_a, inc=1, device_id=(left,),
                    device_id_type=pl.DeviceIdType.MESH,
                )
                pl.semaphore_signal(
                    credit_b, inc=1, device_id=(right,),
                    device_id_type=pl.DeviceIdType.MESH,
                )

        ag_store(N_DEV - 2)

    return pl.pallas_call(
        body,
        out_shape=jax.ShapeDtypeStruct((B, S, C), jnp.float32),
        in_specs=[
            pl.BlockSpec(memory_space=pl.ANY),
            pl.BlockSpec(memory_space=pltpu.VMEM),
            pl.BlockSpec(memory_space=pltpu.VMEM),
        ],
        out_specs=pl.BlockSpec(memory_space=pl.ANY),
        scratch_shapes=[
            pltpu.VMEM((HB, CHUNK + HALO, C), jnp.float32),
            pltpu.VMEM((C, C), jnp.bfloat16),
            pltpu.VMEM((2, HB, CHUNK, C), jnp.float32),
            pltpu.VMEM((2, HB, CHUNK, C), jnp.float32),
            pltpu.VMEM((HB, CHUNK, C), jnp.bfloat16),
            pltpu.VMEM((HB, CHUNK, C), jnp.bfloat16),
            pltpu.VMEM((HB, CHUNK, C), jnp.bfloat16),
            pltpu.VMEM((HB, CHUNK, C), jnp.bfloat16),
            pltpu.VMEM((2, HB, CHUNK, C), jnp.bfloat16),
            pltpu.VMEM((2, HB, CHUNK, C), jnp.bfloat16),
            pltpu.VMEM((HB, CHUNK, C), jnp.float32),
            pltpu.VMEM((HB, CHUNK, C), jnp.float32),
            pltpu.SemaphoreType.DMA,
            pltpu.SemaphoreType.DMA,
            pltpu.SemaphoreType.DMA,
            pltpu.SemaphoreType.DMA,
            pltpu.SemaphoreType.DMA,
            pltpu.SemaphoreType.DMA,
            pltpu.SemaphoreType.DMA,
            pltpu.SemaphoreType.REGULAR,
            pltpu.SemaphoreType.REGULAR,
        ],
        compiler_params=pltpu.CompilerParams(
            collective_id=0, vmem_limit_bytes=56 * 1024 * 1024
        ),
    )(x, k, Wp)


# device time: 233015 ns/iter; 3.6273x vs baseline; 1.0154x over previous
import jax
import jax.numpy as jnp
from jax import lax
from jax.experimental import pallas as pl
from jax.experimental.pallas import tpu as pltpu

N_DEV = 8
B, S, C = 4, 2048, 1024
TAPS = 4
CHUNK = S // N_DEV
HB = B // 2
HALO = 8


def kernel(x, k, Wp):
    def body(
        x_ref, k_ref, w_ref, out_ref,
        xbuf, wbuf, pbuf_a, pbuf_b, rs_recv_a, rs_recv_b, sbuf_a, sbuf_b,
        agr_a, agr_b, stg_a, stg_b,
        xcp_sem, hcp_sem, store_sem,
        send_sem_a, recv_sem_a, send_sem_b, recv_sem_b,
        credit_a, credit_b,
    ):
        def gray(i):
            return jnp.where(i < 4, i, 11 - i)

        mesh_id = lax.axis_index("i")
        my = gray(mesh_id)
        left = gray(lax.rem(my + N_DEV - 1, N_DEV))
        right = gray(lax.rem(my + 1, N_DEV))

        def compute_half(c_idx, boff, dst):
            base = pl.multiple_of(c_idx * CHUNK, CHUNK)
            hstart = pl.multiple_of(jnp.maximum(base - HALO, 0), HALO)
            cp = pltpu.make_async_copy(
                x_ref.at[pl.ds(boff, HB), pl.ds(base, CHUNK), :],
                xbuf.at[:, pl.ds(HALO, CHUNK), :],
                xcp_sem,
            )
            hp = pltpu.make_async_copy(
                x_ref.at[pl.ds(boff, HB), pl.ds(hstart, HALO), :],
                xbuf.at[:, pl.ds(0, HALO), :],
                hcp_sem,
            )
            cp.start()
            hp.start()
            cp.wait()
            hp.wait()
            halo_mask = jnp.where(c_idx == 0, 0.0, 1.0)
            xbuf[:, :HALO, :] = xbuf[:, :HALO, :] * halo_mask
            acc = xbuf[:, pl.ds(HALO, CHUNK), :] * k_ref[TAPS - 1, :][None, None, :]
            for d in range(1, TAPS):
                tap = k_ref[TAPS - 1 - d, :][None, None, :]
                acc = acc + xbuf[:, pl.ds(HALO - d, CHUNK), :] * tap
            a = (acc * jax.nn.sigmoid(acc)).astype(jnp.bfloat16)
            for b in range(HB):
                dst[b, :, :] = jnp.dot(
                    a[b], wbuf[:, :], preferred_element_type=jnp.float32
                )

        barrier = pltpu.get_barrier_semaphore()
        for nbr in (left, right):
            pl.semaphore_signal(
                barrier, inc=1, device_id=(nbr,),
                device_id_type=pl.DeviceIdType.MESH,
            )
        pl.semaphore_wait(barrier, 2)

        pl.semaphore_signal(
            credit_a, inc=1, device_id=(left,),
            device_id_type=pl.DeviceIdType.MESH,
        )
        pl.semaphore_signal(
            credit_b, inc=1, device_id=(right,),
            device_id_type=pl.DeviceIdType.MESH,
        )

        wbuf[:, :] = w_ref[:, :].astype(jnp.bfloat16)

        compute_half(my, 0, pbuf_a.at[0])
        compute_half(my, HB, pbuf_b.at[0])
        sbuf_a[:, :, :] = pbuf_a[0, :, :, :].astype(jnp.bfloat16)
        sbuf_b[:, :, :] = pbuf_b[0, :, :, :].astype(jnp.bfloat16)

        for s in range(N_DEV - 1):
            recv_ca = lax.rem(my - s - 1 + 2 * N_DEV, N_DEV)
            recv_cb = lax.rem(my + s + 1 + 2 * N_DEV, N_DEV)
            cur = s % 2
            nxt = (s + 1) % 2

            pl.semaphore_wait(credit_a, 1)
            pl.semaphore_wait(credit_b, 1)
            rdma_a = pltpu.make_async_remote_copy(
                src_ref=sbuf_a,
                dst_ref=rs_recv_a,
                send_sem=send_sem_a,
                recv_sem=recv_sem_a,
                device_id=(right,),
                device_id_type=pl.DeviceIdType.MESH,
            )
            rdma_b = pltpu.make_async_remote_copy(
                src_ref=sbuf_b,
                dst_ref=rs_recv_b,
                send_sem=send_sem_b,
                recv_sem=recv_sem_b,
                device_id=(left,),
                device_id_type=pl.DeviceIdType.MESH,
            )
            rdma_a.start()
            rdma_b.start()

            compute_half(recv_ca, 0, pbuf_a.at[nxt])
            compute_half(recv_cb, HB, pbuf_b.at[nxt])

            rdma_a.wait()
            rdma_b.wait()

            sum_a = pbuf_a[nxt, :, :, :] + rs_recv_a[:, :, :].astype(
                jnp.float32
            )
            sum_b = pbuf_b[nxt, :, :, :] + rs_recv_b[:, :, :].astype(
                jnp.float32
            )
            pbuf_a[nxt, :, :, :] = sum_a
            sbuf_a[:, :, :] = sum_a.astype(jnp.bfloat16)
            pbuf_b[nxt, :, :, :] = sum_b
            sbuf_b[:, :, :] = sum_b.astype(jnp.bfloat16)

            if s < N_DEV - 2:
                pl.semaphore_signal(
                    credit_a, inc=1, device_id=(left,),
                    device_id_type=pl.DeviceIdType.MESH,
                )
                pl.semaphore_signal(
                    credit_b, inc=1, device_id=(right,),
                    device_id_type=pl.DeviceIdType.MESH,
                )

        own_a = pl.multiple_of(lax.rem(my + 1, N_DEV) * CHUNK, CHUNK)
        own_b = pl.multiple_of(lax.rem(my - 1 + N_DEV, N_DEV) * CHUNK, CHUNK)
        fin_a = pltpu.make_async_copy(
            pbuf_a.at[1], out_ref.at[pl.ds(0, HB), pl.ds(own_a, CHUNK), :],
            store_sem,
        )
        fin_b = pltpu.make_async_copy(
            pbuf_b.at[1], out_ref.at[pl.ds(HB, HB), pl.ds(own_b, CHUNK), :],
            store_sem,
        )
        fin_a.start()
        fin_b.start()
        fin_a.wait()
        fin_b.wait()

        pl.semaphore_signal(
            credit_a, inc=2, device_id=(left,),
            device_id_type=pl.DeviceIdType.MESH,
        )
        pl.semaphore_signal(
            credit_b, inc=2, device_id=(right,),
            device_id_type=pl.DeviceIdType.MESH,
        )

        def ag_store(s):
            slot = s % 2
            ra = pl.multiple_of(
                lax.rem(my - s + 2 * N_DEV, N_DEV) * CHUNK, CHUNK
            )
            rb = pl.multiple_of(
                lax.rem(my + s + 2 * N_DEV, N_DEV) * CHUNK, CHUNK
            )
            stg_a[:, :, :] = agr_a[slot, :, :, :].astype(jnp.float32)
            stg_b[:, :, :] = agr_b[slot, :, :, :].astype(jnp.float32)
            st_a = pltpu.make_async_copy(
                stg_a, out_ref.at[pl.ds(0, HB), pl.ds(ra, CHUNK), :],
                store_sem,
            )
            st_b = pltpu.make_async_copy(
                stg_b, out_ref.at[pl.ds(HB, HB), pl.ds(rb, CHUNK), :],
                store_sem,
            )
            st_a.start()
            st_b.start()
            st_a.wait()
            st_b.wait()

        for s in range(N_DEV - 1):
            cur = s % 2
            prv = (s - 1) % 2

            pl.semaphore_wait(credit_a, 1)
            pl.semaphore_wait(credit_b, 1)
            rdma_a = pltpu.make_async_remote_copy(
                src_ref=sbuf_a if s == 0 else agr_a.at[prv],
                dst_ref=agr_a.at[cur],
                send_sem=send_sem_a,
                recv_sem=recv_sem_a,
                device_id=(right,),
                device_id_type=pl.DeviceIdType.MESH,
            )
            rdma_b = pltpu.make_async_remote_copy(
                src_ref=sbuf_b if s == 0 else agr_b.at[prv],
                dst_ref=agr_b.at[cur],
                send_sem=send_sem_b,
                recv_sem=recv_sem_b,
                device_id=(left,),
                device_id_type=pl.DeviceIdType.MESH,
            )
            rdma_a.start()
            rdma_b.start()

            if s >= 1:
                ag_store(s - 1)

            rdma_a.wait_send()
            rdma_b.wait_send()

            if 1 <= s <= 5:
                pl.semaphore_signal(
                    credit_a, inc=1, device_id=(left,),
                    device_id_type=pl.DeviceIdType.MESH,
                )
                pl.semaphore_signal(
                    credit_b, inc=1, device_id=(right,),
                    device_id_type=pl.DeviceIdType.MESH,
                )

            rdma_a.wait_recv()
            rdma_b.wait_recv()

        ag_store(N_DEV - 2)

    return pl.pallas_call(
        body,
        out_shape=jax.ShapeDtypeStruct((B, S, C), jnp.float32),
        in_specs=[
            pl.BlockSpec(memory_space=pl.ANY),
            pl.BlockSpec(memory_space=pltpu.VMEM),
            pl.BlockSpec(memory_space=pltpu.VMEM),
        ],
        out_specs=pl.BlockSpec(memory_space=pl.ANY),
        scratch_shapes=[
            pltpu.VMEM((HB, CHUNK + HALO, C), jnp.float32),
            pltpu.VMEM((C, C), jnp.bfloat16),
            pltpu.VMEM((2, HB, CHUNK, C), jnp.float32),
            pltpu.VMEM((2, HB, CHUNK, C), jnp.float32),
            pltpu.VMEM((HB, CHUNK, C), jnp.bfloat16),
            pltpu.VMEM((HB, CHUNK, C), jnp.bfloat16),
            pltpu.VMEM((HB, CHUNK, C), jnp.bfloat16),
            pltpu.VMEM((HB, CHUNK, C), jnp.bfloat16),
            pltpu.VMEM((2, HB, CHUNK, C), jnp.bfloat16),
            pltpu.VMEM((2, HB, CHUNK, C), jnp.bfloat16),
            pltpu.VMEM((HB, CHUNK, C), jnp.float32),
            pltpu.VMEM((HB, CHUNK, C), jnp.float32),
            pltpu.SemaphoreType.DMA,
            pltpu.SemaphoreType.DMA,
            pltpu.SemaphoreType.DMA,
            pltpu.SemaphoreType.DMA,
            pltpu.SemaphoreType.DMA,
            pltpu.SemaphoreType.DMA,
            pltpu.SemaphoreType.DMA,
            pltpu.SemaphoreType.REGULAR,
            pltpu.SemaphoreType.REGULAR,
        ],
        compiler_params=pltpu.CompilerParams(
            collective_id=0, vmem_limit_bytes=56 * 1024 * 1024
        ),
    )(x, k, Wp)
